# baseline (speedup 1.0000x reference)
import functools

import jax
import jax.numpy as jnp
from jax.experimental import pallas as pl
from jax.experimental.pallas import tpu as pltpu

_CHUNK = 1024
_NBUF = 8


def _dist_pipeline(x_hbm, c_ref, o_hbm, buf, obuf, insem, outsem):
    n_chunks = x_hbm.shape[0] // _CHUNK

    def _copy_in(chunk, slot):
        return pltpu.make_async_copy(
            x_hbm.at[pl.ds(chunk * _CHUNK, _CHUNK), :],
            buf.at[slot],
            insem.at[slot],
        )

    def _copy_out(chunk, slot):
        return pltpu.make_async_copy(
            obuf.at[slot],
            o_hbm.at[:, pl.ds(chunk * _CHUNK, _CHUNK)],
            outsem.at[slot],
        )

    for s in range(_NBUF):
        _copy_in(s, s).start()

    c = c_ref[...]
    y2 = jnp.sum(c * c, axis=1, keepdims=True)

    def _step(i, carry):
        slot = jax.lax.rem(i, _NBUF)
        _copy_in(i, slot).wait()

        @pl.when(i >= _NBUF)
        def _wait_out():
            _copy_out(i - _NBUF, slot).wait()

        x = buf[slot]
        x2 = jnp.sum(x * x, axis=1)[None, :]
        cx = jax.lax.dot_general(
            c, x, (((1,), (1,)), ((), ())),
            preferred_element_type=jnp.float32,
        )
        obuf[slot] = jnp.maximum(x2 + y2 - 2.0 * cx, 0.0)
        _copy_out(i, slot).start()

        @pl.when(i + _NBUF < n_chunks)
        def _prefetch():
            _copy_in(i + _NBUF, slot).start()

        return carry

    jax.lax.fori_loop(0, n_chunks, _step, 0, unroll=2)
    for s in range(_NBUF):
        chunk = n_chunks - _NBUF + s
        _copy_out(chunk, chunk % _NBUF).wait()


@functools.partial(jax.jit, static_argnames=("interpret",))
def kernel(data, components, labels, interpret=False):
    B, D = data.shape
    K = components.shape[0]
    dist_t = pl.pallas_call(
        _dist_pipeline,
        in_specs=[
            pl.BlockSpec(memory_space=pl.ANY),
            pl.BlockSpec(memory_space=pltpu.VMEM),
        ],
        out_specs=pl.BlockSpec(memory_space=pl.ANY),
        out_shape=jax.ShapeDtypeStruct((K, B), jnp.float32),
        scratch_shapes=[
            pltpu.VMEM((_NBUF, _CHUNK, D), jnp.float32),
            pltpu.VMEM((_NBUF, K, _CHUNK), jnp.float32),
            pltpu.SemaphoreType.DMA((_NBUF,)),
            pltpu.SemaphoreType.DMA((_NBUF,)),
        ],
        interpret=interpret,
    )(data, components)
    return (dist_t.T, labels)


# speedup vs baseline: 1.3402x; 1.3402x over previous
import functools

import jax
import jax.numpy as jnp
from jax.experimental import pallas as pl
from jax.experimental.pallas import tpu as pltpu

_R = 8
_CHUNK = 128
_NBUF = 8


def _dist_pipeline(x_hbm, c2_ref, p_ref, yb_ref, o_hbm, buf, obuf,
                   insem, outsem):
    n_chunks = x_hbm.shape[0] // _CHUNK
    RD = x_hbm.shape[1]
    D = RD // _R

    def _copy_in(chunk, slot):
        return pltpu.make_async_copy(
            x_hbm.at[pl.ds(chunk * _CHUNK, _CHUNK), :],
            buf.at[slot],
            insem.at[slot],
        )

    def _copy_out(chunk, slot):
        return pltpu.make_async_copy(
            obuf.at[slot],
            o_hbm.at[pl.ds(chunk * _CHUNK, _CHUNK), :],
            outsem.at[slot],
        )

    for s in range(_NBUF):
        _copy_in(s, s).start()

    c2 = c2_ref[...]
    p = p_ref[...]
    yb = yb_ref[...]

    def _step(i, carry):
        slot = jax.lax.rem(i, _NBUF)
        _copy_in(i, slot).wait()

        @pl.when(i >= _NBUF)
        def _wait_out():
            _copy_out(i - _NBUF, slot).wait()

        x = buf[slot]
        s8 = jnp.sum(
            jnp.reshape(x * x, (_CHUNK, _R, D)), axis=2)
        x2 = jax.lax.dot_general(
            s8, p, (((1,), (0,)), ((), ())),
            preferred_element_type=jnp.float32,
        )
        cx = jax.lax.dot_general(
            x, c2, (((1,), (0,)), ((), ())),
            preferred_element_type=jnp.float32,
        )
        obuf[slot] = jnp.maximum(x2 + cx + yb, 0.0)
        _copy_out(i, slot).start()

        @pl.when(i + _NBUF < n_chunks)
        def _prefetch():
            _copy_in(i + _NBUF, slot).start()

        return carry

    jax.lax.fori_loop(0, n_chunks, _step, 0, unroll=2)
    for s in range(_NBUF):
        chunk = n_chunks - _NBUF + s
        _copy_out(chunk, chunk % _NBUF).wait()


@functools.partial(jax.jit, static_argnames=("interpret",))
def kernel(data, components, labels, interpret=False):
    B, D = data.shape
    K = components.shape[0]
    xr = jnp.reshape(data, (B // _R, _R * D))
    eye = jnp.eye(_R, dtype=jnp.float32)
    c2 = jnp.reshape(
        eye[:, None, :, None] * (-2.0 * components.T)[None, :, None, :],
        (_R * D, _R * K))
    p = jnp.reshape(
        eye[:, :, None] * jnp.ones((1, 1, K), jnp.float32), (_R, _R * K))
    yb = jnp.tile(jnp.sum(components * components, axis=1), _R)[None, :]

    packed = pl.pallas_call(
        _dist_pipeline,
        in_specs=[
            pl.BlockSpec(memory_space=pl.ANY),
            pl.BlockSpec(memory_space=pltpu.VMEM),
            pl.BlockSpec(memory_space=pltpu.VMEM),
            pl.BlockSpec(memory_space=pltpu.VMEM),
        ],
        out_specs=pl.BlockSpec(memory_space=pl.ANY),
        out_shape=jax.ShapeDtypeStruct((B // _R, _R * K), jnp.float32),
        scratch_shapes=[
            pltpu.VMEM((_NBUF, _CHUNK, _R * D), jnp.float32),
            pltpu.VMEM((_NBUF, _CHUNK, _R * K), jnp.float32),
            pltpu.SemaphoreType.DMA((_NBUF,)),
            pltpu.SemaphoreType.DMA((_NBUF,)),
        ],
        interpret=interpret,
    )(xr, c2, p, yb)
    return (jnp.reshape(packed, (B, K)), labels)


# speedup vs baseline: 1.3709x; 1.0229x over previous
import functools

import jax
import jax.numpy as jnp
from jax.experimental import pallas as pl
from jax.experimental.pallas import tpu as pltpu

_R = 8
_CHUNK = 128
_NBUF = 8


def _dist_pipeline(x_hbm, c2_ref, ones_ref, yb_ref, o_hbm, buf, obuf,
                   insem, outsem):
    n_chunks = x_hbm.shape[0] // _CHUNK

    def _copy_in(chunk, slot):
        return pltpu.make_async_copy(
            x_hbm.at[pl.ds(chunk * _CHUNK, _CHUNK), :],
            buf.at[slot],
            insem.at[slot],
        )

    def _copy_out(chunk, slot):
        return pltpu.make_async_copy(
            obuf.at[slot],
            o_hbm.at[pl.ds(chunk * _CHUNK, _CHUNK), :],
            outsem.at[slot],
        )

    for s in range(_NBUF):
        _copy_in(s, s).start()

    c2 = c2_ref[...]
    ones_bd = ones_ref[...]
    yb = yb_ref[...]

    def _step(i, carry):
        slot = jax.lax.rem(i, _NBUF)
        _copy_in(i, slot).wait()

        @pl.when(i >= _NBUF)
        def _wait_out():
            _copy_out(i - _NBUF, slot).wait()

        x = buf[slot]
        x2 = jax.lax.dot_general(
            x * x, ones_bd, (((1,), (0,)), ((), ())),
            preferred_element_type=jnp.float32,
        )
        cx = jax.lax.dot_general(
            x, c2, (((1,), (0,)), ((), ())),
            preferred_element_type=jnp.float32,
        )
        obuf[slot] = jnp.maximum(x2 + cx + yb, 0.0)
        _copy_out(i, slot).start()

        @pl.when(i + _NBUF < n_chunks)
        def _prefetch():
            _copy_in(i + _NBUF, slot).start()

        return carry

    jax.lax.fori_loop(0, n_chunks, _step, 0, unroll=2)
    for s in range(_NBUF):
        chunk = n_chunks - _NBUF + s
        _copy_out(chunk, chunk % _NBUF).wait()


@functools.partial(jax.jit, static_argnames=("interpret",))
def kernel(data, components, labels, interpret=False):
    B, D = data.shape
    K = components.shape[0]
    xr = jnp.reshape(data, (B // _R, _R * D))
    eye = jnp.eye(_R, dtype=jnp.float32)
    c2 = jnp.reshape(
        eye[:, None, :, None] * (-2.0 * components.T)[None, :, None, :],
        (_R * D, _R * K))
    ones_bd = jnp.reshape(
        eye[:, None, :, None] * jnp.ones((1, D, 1, K), jnp.float32),
        (_R * D, _R * K))
    yb = jnp.tile(jnp.sum(components * components, axis=1), _R)[None, :]

    packed = pl.pallas_call(
        _dist_pipeline,
        in_specs=[
            pl.BlockSpec(memory_space=pl.ANY),
            pl.BlockSpec(memory_space=pltpu.VMEM),
            pl.BlockSpec(memory_space=pltpu.VMEM),
            pl.BlockSpec(memory_space=pltpu.VMEM),
        ],
        out_specs=pl.BlockSpec(memory_space=pl.ANY),
        out_shape=jax.ShapeDtypeStruct((B // _R, _R * K), jnp.float32),
        scratch_shapes=[
            pltpu.VMEM((_NBUF, _CHUNK, _R * D), jnp.float32),
            pltpu.VMEM((_NBUF, _CHUNK, _R * K), jnp.float32),
            pltpu.SemaphoreType.DMA((_NBUF,)),
            pltpu.SemaphoreType.DMA((_NBUF,)),
        ],
        interpret=interpret,
    )(xr, c2, ones_bd, yb)
    return (jnp.reshape(packed, (B, K)), labels)


# speedup vs baseline: 3.7288x; 2.7199x over previous
import functools

import jax
import jax.numpy as jnp
from jax.experimental import pallas as pl
from jax.experimental.pallas import tpu as pltpu

_KP = 16
_CHUNK = 1024
_NBUF = 8


def _dist_pipeline(x_hbm, c2_ref, ones_ref, yb_ref, o_hbm, buf, obuf,
                   insem, outsem):
    n_chunks = x_hbm.shape[0] // _CHUNK

    def _copy_in(chunk, slot):
        return pltpu.make_async_copy(
            x_hbm.at[pl.ds(chunk * _CHUNK, _CHUNK), :],
            buf.at[slot],
            insem.at[slot],
        )

    def _copy_out(chunk, slot):
        return pltpu.make_async_copy(
            obuf.at[slot],
            o_hbm.at[pl.ds(chunk * _CHUNK, _CHUNK), :],
            outsem.at[slot],
        )

    for s in range(_NBUF):
        _copy_in(s, s).start()

    c2 = c2_ref[...]
    ones_c = ones_ref[...]
    yb = yb_ref[...]

    def _step(i, carry):
        slot = jax.lax.rem(i, _NBUF)
        _copy_in(i, slot).wait()

        @pl.when(i >= _NBUF)
        def _wait_out():
            _copy_out(i - _NBUF, slot).wait()

        x = buf[slot]
        x2 = jax.lax.dot_general(
            x * x, ones_c, (((1,), (0,)), ((), ())),
            preferred_element_type=jnp.float32,
        )
        cx = jax.lax.dot_general(
            x, c2, (((1,), (0,)), ((), ())),
            preferred_element_type=jnp.float32,
        )
        obuf[slot] = jnp.maximum(x2 + cx + yb, 0.0)
        _copy_out(i, slot).start()

        @pl.when(i + _NBUF < n_chunks)
        def _prefetch():
            _copy_in(i + _NBUF, slot).start()

        return carry

    jax.lax.fori_loop(0, n_chunks, _step, 0, unroll=2)
    for s in range(_NBUF):
        chunk = n_chunks - _NBUF + s
        _copy_out(chunk, chunk % _NBUF).wait()


@functools.partial(jax.jit, static_argnames=("interpret",))
def kernel(data, components, labels, interpret=False):
    B, D = data.shape
    K = components.shape[0]
    pad = ((0, 0), (0, _KP - K))
    c2 = jnp.pad(-2.0 * components.T, pad)
    ones_c = jnp.pad(jnp.ones((D, K), jnp.float32), pad)
    yb = jnp.pad(jnp.sum(components * components, axis=1)[None, :],
                 ((0, 0), (0, _KP - K)))

    padded = pl.pallas_call(
        _dist_pipeline,
        in_specs=[
            pl.BlockSpec(memory_space=pl.ANY),
            pl.BlockSpec(memory_space=pltpu.VMEM),
            pl.BlockSpec(memory_space=pltpu.VMEM),
            pl.BlockSpec(memory_space=pltpu.VMEM),
        ],
        out_specs=pl.BlockSpec(memory_space=pl.ANY),
        out_shape=jax.ShapeDtypeStruct((B, _KP), jnp.float32),
        scratch_shapes=[
            pltpu.VMEM((_NBUF, _CHUNK, D), jnp.float32),
            pltpu.VMEM((_NBUF, _CHUNK, _KP), jnp.float32),
            pltpu.SemaphoreType.DMA((_NBUF,)),
            pltpu.SemaphoreType.DMA((_NBUF,)),
        ],
        interpret=interpret,
    )(data, c2, ones_c, yb)
    return (padded[:, :K], labels)
